# Initial kernel scaffold; baseline (speedup 1.0000x reference)
import jax
import jax.numpy as jnp
from jax.experimental import pallas as pl


def kernel(visited_goods_ids, visited_shop_ids, visited_cate_ids, visited_goods_prices, query_word_ids, goods_table, shop_table, cate_table, price_table, query_table, conv_w2, conv_b2, conv_w3, conv_b3, conv_w4, conv_b4, bn1_gamma, bn1_beta, bn1_mean, bn1_var, dense1_w, dense1_b, bn2_gamma, bn2_beta, bn2_mean, bn2_var, dense2_w, dense2_b):
    raise NotImplementedError("write your pallas kernel here")


# speedup vs baseline: 1.3668x; 1.3668x over previous
import functools

import jax
import jax.numpy as jnp
from jax import lax
from jax.experimental import pallas as pl
from jax.experimental.pallas import tpu as pltpu
from jax.experimental.pallas import tpu_sc as plsc

B, L, Q, D = 4096, 50, 20, 32
FILTERS = 32
F1, F2 = 128, 64
EPS = 1e-6

NW = 32
BLK = 320
SUB = 64
GROUPS = BLK // 16
SEQ_PER_W = (B * L) // NW
SEQ_BLOCKS = SEQ_PER_W // BLK
QRY_PER_W = (B * Q) // NW
QRY_BLOCKS = QRY_PER_W // BLK
CV_ROWS, PV_ROWS = 1000, 100


def _sc_gather_body(gids, sids, cids, pids, qids,
                    gtab4, stab4, qtab4, ctab4, ptab4,
                    g_out, s_out, c_out, p_out, q_out,
                    idx_v, gidx_v, wide_v, out_v, cate_v, price_v, sem):
    wid = lax.axis_index("s") * 2 + lax.axis_index("c")
    lanes = lax.iota(jnp.int32, 16)

    pltpu.sync_copy(ctab4, cate_v)
    pltpu.sync_copy(ptab4, price_v)

    def extract(i, src_v, src_rows16, idx16, out_off):
        o = i * 16
        rows = lanes + o
        cbase = (idx16 & 3) * 32
        for w in range(D):
            vals = plsc.load_gather(src_v, [src_rows16, cbase + w])
            flat = (rows + out_off) * 32 + w
            plsc.store_scatter(out_v,
                               [lax.shift_right_logical(flat, 7), flat & 127],
                               vals)

    def wide_block(ids_hbm, tab4_hbm, out_hbm, base, base4):
        pltpu.sync_copy(ids_hbm.at[pl.ds(base, BLK)], idx_v)

        def shift_body(i, carry):
            o = i * 16
            gidx_v[pl.ds(o, 16)] = lax.shift_right_logical(
                idx_v[pl.ds(o, 16)], 2)
            return carry
        lax.fori_loop(0, GROUPS, shift_body, 0)

        cps = []
        for off in range(0, BLK, SUB):
            cps.append(pltpu.async_copy(
                tab4_hbm.at[gidx_v.at[pl.ds(off, SUB)]],
                wide_v.at[pl.ds(off, SUB)],
                sem))
        for cp in cps:
            cp.wait()

        def ex_body(i, carry):
            o = i * 16
            extract(i, wide_v, lanes + o, idx_v[pl.ds(o, 16)], 0)
            return carry
        lax.fori_loop(0, GROUPS, ex_body, 0)
        pltpu.sync_copy(out_v, out_hbm.at[pl.ds(base4, BLK // 4)])

    def small_block(ids_hbm, tab_v, out_hbm, base, base4):
        pltpu.sync_copy(ids_hbm.at[pl.ds(base, BLK)], idx_v)

        def ex_body(i, carry):
            o = i * 16
            idx16 = idx_v[pl.ds(o, 16)]
            extract(i, tab_v, lax.shift_right_logical(idx16, 2), idx16, 0)
            return carry
        lax.fori_loop(0, GROUPS, ex_body, 0)
        pltpu.sync_copy(out_v, out_hbm.at[pl.ds(base4, BLK // 4)])

    def gbody(i, carry):
        wide_block(gids, gtab4, g_out, wid * SEQ_PER_W + i * BLK,
                   wid * (SEQ_PER_W // 4) + i * (BLK // 4))
        return carry
    lax.fori_loop(0, SEQ_BLOCKS, gbody, 0)

    def sbody(i, carry):
        wide_block(sids, stab4, s_out, wid * SEQ_PER_W + i * BLK,
                   wid * (SEQ_PER_W // 4) + i * (BLK // 4))
        return carry
    lax.fori_loop(0, SEQ_BLOCKS, sbody, 0)

    def cbody(i, carry):
        small_block(cids, cate_v, c_out, wid * SEQ_PER_W + i * BLK,
                    wid * (SEQ_PER_W // 4) + i * (BLK // 4))
        return carry
    lax.fori_loop(0, SEQ_BLOCKS, cbody, 0)

    def pbody(i, carry):
        small_block(pids, price_v, p_out, wid * SEQ_PER_W + i * BLK,
                    wid * (SEQ_PER_W // 4) + i * (BLK // 4))
        return carry
    lax.fori_loop(0, SEQ_BLOCKS, pbody, 0)

    def qbody(i, carry):
        wide_block(qids, qtab4, q_out, wid * QRY_PER_W + i * BLK,
                   wid * (QRY_PER_W // 4) + i * (BLK // 4))
        return carry
    lax.fori_loop(0, QRY_BLOCKS, qbody, 0)


@functools.cache
def _make_sc_gather():
    return pl.kernel(
        _sc_gather_body,
        out_type=[jax.ShapeDtypeStruct((L * B // 4, 128), jnp.float32),
                  jax.ShapeDtypeStruct((L * B // 4, 128), jnp.float32),
                  jax.ShapeDtypeStruct((L * B // 4, 128), jnp.float32),
                  jax.ShapeDtypeStruct((L * B // 4, 128), jnp.float32),
                  jax.ShapeDtypeStruct((Q * B // 4, 128), jnp.float32)],
        mesh=plsc.VectorSubcoreMesh(core_axis_name="c", subcore_axis_name="s"),
        scratch_types=[pltpu.VMEM((BLK,), jnp.int32),
                       pltpu.VMEM((BLK,), jnp.int32),
                       pltpu.VMEM((BLK, 128), jnp.float32),
                       pltpu.VMEM((BLK // 4, 128), jnp.float32),
                       pltpu.VMEM((CV_ROWS // 4, 128), jnp.float32),
                       pltpu.VMEM((PV_ROWS // 4, 128), jnp.float32),
                       pltpu.SemaphoreType.DMA],
        compiler_params=pltpu.CompilerParams(needs_layout_passes=False),
    )


BQ = 256


def _conv_body(qemb_ref, cw_ref, cb_ref, w1q_ref, b1_ref, qw_ref):
    feats = []
    off = 0
    for k in (2, 3, 4):
        t_len = Q - k + 1
        m = None
        for t in range(t_len):
            acc = lax.dot_general(qemb_ref[t], cw_ref[off],
                                  (((1,), (0,)), ((), ())))
            for j in range(1, k):
                acc = acc + lax.dot_general(qemb_ref[t + j], cw_ref[off + j],
                                            (((1,), (0,)), ((), ())))
            m = acc if m is None else jnp.maximum(m, acc)
        off += k
        feats.append(jax.nn.relu(m + cb_ref[k - 2]))
    qfeat = jnp.concatenate(feats, axis=-1)
    qw_ref[...] = lax.dot_general(qfeat, w1q_ref[...],
                                  (((1,), (0,)), ((), ()))) + b1_ref[...]


_tc_conv = pl.pallas_call(
    _conv_body,
    grid=(B // BQ,),
    in_specs=[
        pl.BlockSpec((Q, BQ, D), lambda i: (0, i, 0)),
        pl.BlockSpec((9, D, FILTERS), lambda i: (0, 0, 0)),
        pl.BlockSpec((3, FILTERS), lambda i: (0, 0)),
        pl.BlockSpec((96, F1), lambda i: (0, 0)),
        pl.BlockSpec((1, F1), lambda i: (0, 0)),
    ],
    out_specs=pl.BlockSpec((BQ, F1), lambda i: (i, 0)),
    out_shape=jax.ShapeDtypeStruct((B, F1), jnp.float32),
)


BB = 512


def _mlp_body(g_ref, s_ref, c_ref, p_ref, qw_ref,
              w1g_ref, w1s_ref, w1c_ref, w1p_ref, w2_ref, b2_ref, out_ref):
    h1 = qw_ref[...]
    for r, w in ((g_ref, w1g_ref), (s_ref, w1s_ref),
                 (c_ref, w1c_ref), (p_ref, w1p_ref)):
        h1 = h1 + lax.dot_general(r[0], w[...], (((1,), (0,)), ((), ())))
    h1 = jax.nn.relu(h1)
    h2 = lax.dot_general(h1, w2_ref[...], (((1,), (0,)), ((), ())))
    out_ref[0] = jax.nn.relu(h2 + b2_ref[...])


_tc_mlp = pl.pallas_call(
    _mlp_body,
    grid=(B // BB, L),
    in_specs=[
        pl.BlockSpec((1, BB, D), lambda i, l: (l, i, 0)),
        pl.BlockSpec((1, BB, D), lambda i, l: (l, i, 0)),
        pl.BlockSpec((1, BB, D), lambda i, l: (l, i, 0)),
        pl.BlockSpec((1, BB, D), lambda i, l: (l, i, 0)),
        pl.BlockSpec((BB, F1), lambda i, l: (i, 0)),
        pl.BlockSpec((D, F1), lambda i, l: (0, 0)),
        pl.BlockSpec((D, F1), lambda i, l: (0, 0)),
        pl.BlockSpec((D, F1), lambda i, l: (0, 0)),
        pl.BlockSpec((D, F1), lambda i, l: (0, 0)),
        pl.BlockSpec((F1, F2), lambda i, l: (0, 0)),
        pl.BlockSpec((1, F2), lambda i, l: (0, 0)),
    ],
    out_specs=pl.BlockSpec((1, BB, F2), lambda i, l: (l, i, 0)),
    out_shape=jax.ShapeDtypeStruct((L, B, F2), jnp.float32),
)


def kernel(visited_goods_ids, visited_shop_ids, visited_cate_ids,
           visited_goods_prices, query_word_ids, goods_table, shop_table,
           cate_table, price_table, query_table,
           conv_w2, conv_b2, conv_w3, conv_b3, conv_w4, conv_b4,
           bn1_gamma, bn1_beta, bn1_mean, bn1_var, dense1_w, dense1_b,
           bn2_gamma, bn2_beta, bn2_mean, bn2_var, dense2_w, dense2_b):
    gi = visited_goods_ids.T.reshape(-1).astype(jnp.int32)
    si = visited_shop_ids.T.reshape(-1).astype(jnp.int32)
    ci = visited_cate_ids.T.reshape(-1).astype(jnp.int32)
    pi = visited_goods_prices.T.reshape(-1).astype(jnp.int32)
    qi = query_word_ids.T.reshape(-1).astype(jnp.int32)

    ge, se, ce, pe, qe = _make_sc_gather()(
        gi, si, ci, pi, qi,
        goods_table.reshape(-1, 128), shop_table.reshape(-1, 128),
        query_table.reshape(-1, 128), cate_table.reshape(-1, 128),
        price_table.reshape(-1, 128))

    scale1 = bn1_gamma * lax.rsqrt(bn1_var + EPS)
    w1f = dense1_w * scale1[:, None]
    b1f = dense1_b + (bn1_beta - bn1_mean * scale1) @ dense1_w
    scale2 = bn2_gamma * lax.rsqrt(bn2_var + EPS)
    w2f = dense2_w * scale2[:, None]
    b2f = dense2_b + (bn2_beta - bn2_mean * scale2) @ dense2_w

    cw = jnp.concatenate([conv_w2, conv_w3, conv_w4], axis=0)
    cb = jnp.stack([conv_b2, conv_b3, conv_b4])

    qw = _tc_conv(qe.reshape(Q, B, D), cw, cb, w1f[4 * D:],
                  b1f.reshape(1, F1))
    x_lb = _tc_mlp(ge.reshape(L, B, D), se.reshape(L, B, D),
                   ce.reshape(L, B, D), pe.reshape(L, B, D), qw,
                   w1f[0 * D:1 * D], w1f[1 * D:2 * D],
                   w1f[2 * D:3 * D], w1f[3 * D:4 * D],
                   w2f, b2f.reshape(1, F2))
    x = jnp.transpose(x_lb, (1, 0, 2))
    seq_len = jnp.full((B,), L, dtype=jnp.int32)
    return x, seq_len


# speedup vs baseline: 1.4374x; 1.0517x over previous
import functools

import jax
import jax.numpy as jnp
from jax import lax
from jax.experimental import pallas as pl
from jax.experimental.pallas import tpu as pltpu
from jax.experimental.pallas import tpu_sc as plsc

B, L, Q, D = 4096, 50, 20, 32
FILTERS = 32
F1, F2 = 128, 64
EPS = 1e-6

NW = 32
BLK = 256
SUB = 128
GROUPS = BLK // 16
OBLK = BLK // 4
SEQ_PER_W = (B * L) // NW
SEQ_BLOCKS = SEQ_PER_W // BLK
QRY_PER_W = (B * Q) // NW
QRY_BLOCKS = QRY_PER_W // BLK
CV_ROWS, PV_ROWS = 1000, 100


def _sc_gather_body(gids, sids, cids, pids, qids,
                    gtab4, stab4, qtab4, ctab4, ptab4,
                    g_out, s_out, c_out, p_out, q_out,
                    idx0, idx1, gidx0, gidx1, wide0, wide1, out_v,
                    cate_v, price_v, sem_g0, sem_g1, sem_w):
    wid = lax.axis_index("s") * 2 + lax.axis_index("c")
    lanes = lax.iota(jnp.int32, 16)
    idx_s = (idx0, idx1)
    gidx_s = (gidx0, gidx1)
    wide_s = (wide0, wide1)
    sem_gs = (sem_g0, sem_g1)

    pltpu.sync_copy(ctab4, cate_v)
    pltpu.sync_copy(ptab4, price_v)

    def load_shift(ids_hbm, off, s):
        pltpu.sync_copy(ids_hbm.at[pl.ds(off, BLK)], idx_s[s])

        def sh(i, c):
            o = i * 16
            gidx_s[s][pl.ds(o, 16)] = lax.shift_right_logical(
                idx_s[s][pl.ds(o, 16)], 2)
            return c
        lax.fori_loop(0, GROUPS, sh, 0)

    def fire(tab4, s):
        for c in range(0, BLK, SUB):
            pltpu.async_copy(tab4.at[gidx_s[s].at[pl.ds(c, SUB)]],
                             wide_s[s].at[pl.ds(c, SUB)], sem_gs[s])

    def drain_g(tab4, s):
        pltpu.make_async_copy(tab4.at[pl.ds(0, BLK)], wide_s[s],
                              sem_gs[s]).wait()

    def drain_w(out_hbm):
        pltpu.make_async_copy(out_v, out_hbm.at[pl.ds(0, OBLK)], sem_w).wait()

    def extract_block(src_v, s, wide):
        def ex(i, c):
            o = i * 16
            rows = lanes + o
            idx16 = idx_s[s][pl.ds(o, 16)]
            srows = rows if wide else lax.shift_right_logical(idx16, 2)
            cbase = (idx16 & 3) * 32
            for w in range(D):
                vals = plsc.load_gather(src_v, [srows, cbase + w])
                flat = rows * 32 + w
                plsc.store_scatter(
                    out_v, [lax.shift_right_logical(flat, 7), flat & 127],
                    vals)
            return c
        lax.fori_loop(0, GROUPS, ex, 0)

    def wide_table(ids_hbm, tab4, out_hbm, nblocks, base, base4):
        load_shift(ids_hbm, base, 0)
        fire(tab4, 0)

        def step(j, s):
            ns = 1 - s

            @pl.when(j + 1 < nblocks)
            def _():
                load_shift(ids_hbm, base + (j + 1) * BLK, ns)
                fire(tab4, ns)
            drain_g(tab4, s)

            @pl.when(j > 0)
            def _():
                drain_w(out_hbm)
            extract_block(wide_s[s], s, True)
            pltpu.async_copy(out_v, out_hbm.at[pl.ds(base4 + j * OBLK, OBLK)],
                             sem_w)

        def body(j, c):
            @pl.when((j & 1) == 0)
            def _():
                step(j, 0)

            @pl.when((j & 1) == 1)
            def _():
                step(j, 1)
            return c
        lax.fori_loop(0, nblocks, body, 0)
        drain_w(out_hbm)

    def small_table(ids_hbm, tab_v, out_hbm, nblocks, base, base4):
        pltpu.sync_copy(ids_hbm.at[pl.ds(base, BLK)], idx_s[0])

        def step(j, s):
            ns = 1 - s

            @pl.when(j + 1 < nblocks)
            def _():
                pltpu.sync_copy(ids_hbm.at[pl.ds(base + (j + 1) * BLK, BLK)],
                                idx_s[ns])

            @pl.when(j > 0)
            def _():
                drain_w(out_hbm)
            extract_block(tab_v, s, False)
            pltpu.async_copy(out_v, out_hbm.at[pl.ds(base4 + j * OBLK, OBLK)],
                             sem_w)

        def body(j, c):
            @pl.when((j & 1) == 0)
            def _():
                step(j, 0)

            @pl.when((j & 1) == 1)
            def _():
                step(j, 1)
            return c
        lax.fori_loop(0, nblocks, body, 0)
        drain_w(out_hbm)

    sbase = wid * SEQ_PER_W
    sbase4 = wid * (SEQ_PER_W // 4)
    wide_table(gids, gtab4, g_out, SEQ_BLOCKS, sbase, sbase4)
    wide_table(sids, stab4, s_out, SEQ_BLOCKS, sbase, sbase4)
    small_table(cids, cate_v, c_out, SEQ_BLOCKS, sbase, sbase4)
    small_table(pids, price_v, p_out, SEQ_BLOCKS, sbase, sbase4)
    wide_table(qids, qtab4, q_out, QRY_BLOCKS, wid * QRY_PER_W,
               wid * (QRY_PER_W // 4))


@functools.cache
def _make_sc_gather():
    return pl.kernel(
        _sc_gather_body,
        out_type=[jax.ShapeDtypeStruct((L * B // 4, 128), jnp.float32),
                  jax.ShapeDtypeStruct((L * B // 4, 128), jnp.float32),
                  jax.ShapeDtypeStruct((L * B // 4, 128), jnp.float32),
                  jax.ShapeDtypeStruct((L * B // 4, 128), jnp.float32),
                  jax.ShapeDtypeStruct((Q * B // 4, 128), jnp.float32)],
        mesh=plsc.VectorSubcoreMesh(core_axis_name="c", subcore_axis_name="s"),
        scratch_types=[pltpu.VMEM((BLK,), jnp.int32),
                       pltpu.VMEM((BLK,), jnp.int32),
                       pltpu.VMEM((BLK,), jnp.int32),
                       pltpu.VMEM((BLK,), jnp.int32),
                       pltpu.VMEM((BLK, 128), jnp.float32),
                       pltpu.VMEM((BLK, 128), jnp.float32),
                       pltpu.VMEM((OBLK, 128), jnp.float32),
                       pltpu.VMEM((CV_ROWS // 4, 128), jnp.float32),
                       pltpu.VMEM((PV_ROWS // 4, 128), jnp.float32),
                       pltpu.SemaphoreType.DMA,
                       pltpu.SemaphoreType.DMA,
                       pltpu.SemaphoreType.DMA],
        compiler_params=pltpu.CompilerParams(needs_layout_passes=False),
    )


BQ = 256


def _conv_body(qemb_ref, cw_ref, cb_ref, w1q_ref, b1_ref, qw_ref):
    feats = []
    off = 0
    for k in (2, 3, 4):
        t_len = Q - k + 1
        m = None
        for t in range(t_len):
            acc = lax.dot_general(qemb_ref[t], cw_ref[off],
                                  (((1,), (0,)), ((), ())))
            for j in range(1, k):
                acc = acc + lax.dot_general(qemb_ref[t + j], cw_ref[off + j],
                                            (((1,), (0,)), ((), ())))
            m = acc if m is None else jnp.maximum(m, acc)
        off += k
        feats.append(jax.nn.relu(m + cb_ref[k - 2]))
    qfeat = jnp.concatenate(feats, axis=-1)
    qw_ref[...] = lax.dot_general(qfeat, w1q_ref[...],
                                  (((1,), (0,)), ((), ()))) + b1_ref[...]


_tc_conv = pl.pallas_call(
    _conv_body,
    grid=(B // BQ,),
    in_specs=[
        pl.BlockSpec((Q, BQ, D), lambda i: (0, i, 0)),
        pl.BlockSpec((9, D, FILTERS), lambda i: (0, 0, 0)),
        pl.BlockSpec((3, FILTERS), lambda i: (0, 0)),
        pl.BlockSpec((96, F1), lambda i: (0, 0)),
        pl.BlockSpec((1, F1), lambda i: (0, 0)),
    ],
    out_specs=pl.BlockSpec((BQ, F1), lambda i: (i, 0)),
    out_shape=jax.ShapeDtypeStruct((B, F1), jnp.float32),
)


BB = 512


def _mlp_body(g_ref, s_ref, c_ref, p_ref, qw_ref,
              w1g_ref, w1s_ref, w1c_ref, w1p_ref, w2_ref, b2_ref, out_ref):
    h1 = qw_ref[...]
    for r, w in ((g_ref, w1g_ref), (s_ref, w1s_ref),
                 (c_ref, w1c_ref), (p_ref, w1p_ref)):
        h1 = h1 + lax.dot_general(r[0], w[...], (((1,), (0,)), ((), ())))
    h1 = jax.nn.relu(h1)
    h2 = lax.dot_general(h1, w2_ref[...], (((1,), (0,)), ((), ())))
    out_ref[0] = jax.nn.relu(h2 + b2_ref[...])


_tc_mlp = pl.pallas_call(
    _mlp_body,
    grid=(B // BB, L),
    in_specs=[
        pl.BlockSpec((1, BB, D), lambda i, l: (l, i, 0)),
        pl.BlockSpec((1, BB, D), lambda i, l: (l, i, 0)),
        pl.BlockSpec((1, BB, D), lambda i, l: (l, i, 0)),
        pl.BlockSpec((1, BB, D), lambda i, l: (l, i, 0)),
        pl.BlockSpec((BB, F1), lambda i, l: (i, 0)),
        pl.BlockSpec((D, F1), lambda i, l: (0, 0)),
        pl.BlockSpec((D, F1), lambda i, l: (0, 0)),
        pl.BlockSpec((D, F1), lambda i, l: (0, 0)),
        pl.BlockSpec((D, F1), lambda i, l: (0, 0)),
        pl.BlockSpec((F1, F2), lambda i, l: (0, 0)),
        pl.BlockSpec((1, F2), lambda i, l: (0, 0)),
    ],
    out_specs=pl.BlockSpec((1, BB, F2), lambda i, l: (l, i, 0)),
    out_shape=jax.ShapeDtypeStruct((L, B, F2), jnp.float32),
)


def kernel(visited_goods_ids, visited_shop_ids, visited_cate_ids,
           visited_goods_prices, query_word_ids, goods_table, shop_table,
           cate_table, price_table, query_table,
           conv_w2, conv_b2, conv_w3, conv_b3, conv_w4, conv_b4,
           bn1_gamma, bn1_beta, bn1_mean, bn1_var, dense1_w, dense1_b,
           bn2_gamma, bn2_beta, bn2_mean, bn2_var, dense2_w, dense2_b):
    gi = visited_goods_ids.T.reshape(-1).astype(jnp.int32)
    si = visited_shop_ids.T.reshape(-1).astype(jnp.int32)
    ci = visited_cate_ids.T.reshape(-1).astype(jnp.int32)
    pi = visited_goods_prices.T.reshape(-1).astype(jnp.int32)
    qi = query_word_ids.T.reshape(-1).astype(jnp.int32)

    ge, se, ce, pe, qe = _make_sc_gather()(
        gi, si, ci, pi, qi,
        goods_table.reshape(-1, 128), shop_table.reshape(-1, 128),
        query_table.reshape(-1, 128), cate_table.reshape(-1, 128),
        price_table.reshape(-1, 128))

    scale1 = bn1_gamma * lax.rsqrt(bn1_var + EPS)
    w1f = dense1_w * scale1[:, None]
    b1f = dense1_b + (bn1_beta - bn1_mean * scale1) @ dense1_w
    scale2 = bn2_gamma * lax.rsqrt(bn2_var + EPS)
    w2f = dense2_w * scale2[:, None]
    b2f = dense2_b + (bn2_beta - bn2_mean * scale2) @ dense2_w

    cw = jnp.concatenate([conv_w2, conv_w3, conv_w4], axis=0)
    cb = jnp.stack([conv_b2, conv_b3, conv_b4])

    qw = _tc_conv(qe.reshape(Q, B, D), cw, cb, w1f[4 * D:],
                  b1f.reshape(1, F1))
    x_lb = _tc_mlp(ge.reshape(L, B, D), se.reshape(L, B, D),
                   ce.reshape(L, B, D), pe.reshape(L, B, D), qw,
                   w1f[0 * D:1 * D], w1f[1 * D:2 * D],
                   w1f[2 * D:3 * D], w1f[3 * D:4 * D],
                   w2f, b2f.reshape(1, F2))
    x = jnp.transpose(x_lb, (1, 0, 2))
    seq_len = jnp.full((B,), L, dtype=jnp.int32)
    return x, seq_len
